# baseline (speedup 1.0000x reference)
import jax
import jax.numpy as jnp
from jax.experimental import pallas as pl
from jax.experimental.pallas import tpu as pltpu

_LEVEL = 15.0
_EPS = 0.001
_HALF_L = (_LEVEL - 1.0) * (1.0 + _EPS) / 2.0
_HALF_W = 7.0


def _fsq_block(z_ref, wct_ref, bc_ref, wet_ref, be_ref, traw_ref,
               zq_ref, err_ref, acc_ref):
    i = pl.program_id(0)
    n = pl.num_programs(0)
    t = jax.nn.softplus(traw_ref[...])
    z = z_ref[...]
    zc = jnp.dot(z, wct_ref[...],
                 preferred_element_type=jnp.float32) + bc_ref[...]
    z_bound = jnp.tanh(zc / t) * _HALF_L
    k = jnp.round(z_bound)
    codes = k * (t * (1.0 / _HALF_W))
    diff = zc - codes
    blk_err = jnp.sum(diff * diff)

    @pl.when(i == 0)
    def _init():
        acc_ref[0] = 0.0

    acc_ref[0] += blk_err

    @pl.when(i == n - 1)
    def _flush():
        err_ref[...] = jnp.full((1, 1, 1), acc_ref[0], jnp.float32)

    k_aug = jnp.concatenate(
        [k, jnp.ones((k.shape[0], 1), jnp.float32)], axis=1)
    we_scaled = wet_ref[...] * (t.reshape(3, 1) * (1.0 / _HALF_W))
    we_aug = jnp.concatenate([we_scaled, be_ref[...]], axis=0)
    zq_ref[...] = jnp.dot(k_aug.astype(jnp.bfloat16),
                          we_aug.astype(jnp.bfloat16),
                          preferred_element_type=jnp.float32)


def kernel(z, W_c, b_c, W_e, b_e, T_raw):
    B, S, D = z.shape
    rows = B * S
    z2 = z.reshape(rows, D)
    blk = 2048
    grid = rows // blk

    zq2, err = pl.pallas_call(
        _fsq_block,
        grid=(grid,),
        in_specs=[
            pl.BlockSpec((blk, D), lambda i: (i, 0)),
            pl.BlockSpec((D, 3), lambda i: (0, 0)),
            pl.BlockSpec((1, 3), lambda i: (0, 0)),
            pl.BlockSpec((3, D), lambda i: (0, 0)),
            pl.BlockSpec((1, D), lambda i: (0, 0)),
            pl.BlockSpec((1, 3), lambda i: (0, 0)),
        ],
        out_specs=[
            pl.BlockSpec((blk, D), lambda i: (i, 0)),
            pl.BlockSpec((1, 1, 1), lambda i: (0, 0, 0)),
        ],
        out_shape=[
            jax.ShapeDtypeStruct((rows, D), jnp.float32),
            jax.ShapeDtypeStruct((1, 1, 1), jnp.float32),
        ],
        scratch_shapes=[pltpu.SMEM((1,), jnp.float32)],
    )(z2, W_c.T, b_c.reshape(1, 3), W_e.T, b_e.reshape(1, D),
      T_raw.reshape(1, 3))

    z_q = zq2.reshape(B, S, D)
    quantization_error = err[0, 0, 0] / (rows * 3)
    return (z_q, quantization_error)


# speedup vs baseline: 1.0064x; 1.0064x over previous
import jax
import jax.numpy as jnp
from jax.experimental import pallas as pl
from jax.experimental.pallas import tpu as pltpu

_LEVEL = 15.0
_EPS = 0.001
_HALF_L = (_LEVEL - 1.0) * (1.0 + _EPS) / 2.0
_HALF_W = 7.0


def _fsq_block(z_ref, wct_ref, bc_ref, wet_ref, be_ref, traw_ref,
               zq_ref, err_ref, acc_ref):
    i = pl.program_id(0)
    n = pl.num_programs(0)
    t = jax.nn.softplus(traw_ref[...])
    z = z_ref[...]
    zc = jnp.dot(z, wct_ref[...],
                 preferred_element_type=jnp.float32) + bc_ref[...]
    z_bound = jnp.tanh(zc / t) * _HALF_L
    k = jnp.round(z_bound)
    codes = k * (t * (1.0 / _HALF_W))
    diff = zc - codes
    blk_err = jnp.sum(diff * diff)

    @pl.when(i == 0)
    def _init():
        acc_ref[0] = 0.0

    acc_ref[0] += blk_err

    @pl.when(i == n - 1)
    def _flush():
        err_ref[...] = jnp.full((1, 1, 1), acc_ref[0], jnp.float32)

    k_aug = jnp.concatenate(
        [k, jnp.ones((k.shape[0], 1), jnp.float32)], axis=1)
    we_scaled = wet_ref[...] * (t.reshape(3, 1) * (1.0 / _HALF_W))
    we_aug = jnp.concatenate([we_scaled, be_ref[...]], axis=0)
    zq_ref[...] = jnp.dot(k_aug.astype(jnp.bfloat16),
                          we_aug.astype(jnp.bfloat16),
                          preferred_element_type=jnp.float32)


def kernel(z, W_c, b_c, W_e, b_e, T_raw):
    B, S, D = z.shape
    rows = B * S
    z2 = z.reshape(rows, D)
    blk = 2048
    grid = rows // blk

    zq2, err = pl.pallas_call(
        _fsq_block,
        grid=(grid,),
        in_specs=[
            pl.BlockSpec((blk, D), lambda i: (i, 0)),
            pl.BlockSpec((D, 3), lambda i: (0, 0)),
            pl.BlockSpec((1, 3), lambda i: (0, 0)),
            pl.BlockSpec((3, D), lambda i: (0, 0)),
            pl.BlockSpec((1, D), lambda i: (0, 0)),
            pl.BlockSpec((1, 3), lambda i: (0, 0)),
        ],
        out_specs=[
            pl.BlockSpec((blk, D), lambda i: (i, 0)),
            pl.BlockSpec((1, 1, 1), lambda i: (0, 0, 0)),
        ],
        out_shape=[
            jax.ShapeDtypeStruct((rows, D), jnp.float32),
            jax.ShapeDtypeStruct((1, 1, 1), jnp.float32),
        ],
        scratch_shapes=[pltpu.SMEM((1,), jnp.float32)],
        compiler_params=pltpu.CompilerParams(
            dimension_semantics=("parallel",),
        ),
    )(z2, W_c.T, b_c.reshape(1, 3), W_e.T, b_e.reshape(1, D),
      T_raw.reshape(1, 3))

    z_q = zq2.reshape(B, S, D)
    quantization_error = err[0, 0, 0] / (rows * 3)
    return (z_q, quantization_error)


# speedup vs baseline: 1.0074x; 1.0010x over previous
import jax
import jax.numpy as jnp
from jax.experimental import pallas as pl
from jax.experimental.pallas import tpu as pltpu

_LEVEL = 15.0
_EPS = 0.001
_HALF_L = (_LEVEL - 1.0) * (1.0 + _EPS) / 2.0
_HALF_W = 7.0


def _fsq_block(z_ref, wct_ref, bc_ref, wet_ref, be_ref, traw_ref,
               zq_ref, err_ref, acc_ref):
    i = pl.program_id(0)
    n = pl.num_programs(0)
    t = jax.nn.softplus(traw_ref[...])
    z = z_ref[...]
    zc = jnp.dot(z, wct_ref[...],
                 preferred_element_type=jnp.float32) + bc_ref[...]
    z_bound = jnp.tanh(zc / t) * _HALF_L
    k = jnp.round(z_bound)
    codes = k * (t * (1.0 / _HALF_W))
    diff = zc - codes
    part = jnp.sum(diff * diff, axis=0, keepdims=True)

    @pl.when(i == 0)
    def _init():
        acc_ref[...] = jnp.zeros((1, 3), jnp.float32)

    acc_ref[...] += part

    @pl.when(i == n - 1)
    def _flush():
        err_ref[...] = jnp.full((1, 1, 1), jnp.sum(acc_ref[...]),
                                jnp.float32)

    k_aug = jnp.concatenate(
        [k, jnp.ones((k.shape[0], 1), jnp.float32)], axis=1)
    we_scaled = wet_ref[...] * (t.reshape(3, 1) * (1.0 / _HALF_W))
    we_aug = jnp.concatenate([we_scaled, be_ref[...]], axis=0)
    zq_ref[...] = jnp.dot(k_aug.astype(jnp.bfloat16),
                          we_aug.astype(jnp.bfloat16),
                          preferred_element_type=jnp.float32)


def kernel(z, W_c, b_c, W_e, b_e, T_raw):
    B, S, D = z.shape
    rows = B * S
    z2 = z.reshape(rows, D)
    blk = 2048
    grid = rows // blk

    zq2, err = pl.pallas_call(
        _fsq_block,
        grid=(grid,),
        in_specs=[
            pl.BlockSpec((blk, D), lambda i: (i, 0)),
            pl.BlockSpec((D, 3), lambda i: (0, 0)),
            pl.BlockSpec((1, 3), lambda i: (0, 0)),
            pl.BlockSpec((3, D), lambda i: (0, 0)),
            pl.BlockSpec((1, D), lambda i: (0, 0)),
            pl.BlockSpec((1, 3), lambda i: (0, 0)),
        ],
        out_specs=[
            pl.BlockSpec((blk, D), lambda i: (i, 0)),
            pl.BlockSpec((1, 1, 1), lambda i: (0, 0, 0)),
        ],
        out_shape=[
            jax.ShapeDtypeStruct((rows, D), jnp.float32),
            jax.ShapeDtypeStruct((1, 1, 1), jnp.float32),
        ],
        scratch_shapes=[pltpu.VMEM((1, 3), jnp.float32)],
        compiler_params=pltpu.CompilerParams(
            dimension_semantics=("parallel",),
        ),
    )(z2, W_c.T, b_c.reshape(1, 3), W_e.T, b_e.reshape(1, D),
      T_raw.reshape(1, 3))

    z_q = zq2.reshape(B, S, D)
    quantization_error = err[0, 0, 0] / (rows * 3)
    return (z_q, quantization_error)


# speedup vs baseline: 1.0563x; 1.0486x over previous
import jax
import jax.numpy as jnp
from jax.experimental import pallas as pl
from jax.experimental.pallas import tpu as pltpu

_LEVEL = 15.0
_EPS = 0.001
_HALF_L = (_LEVEL - 1.0) * (1.0 + _EPS) / 2.0
_HALF_W = 7.0


def _fsq_block(z_ref, wct_ref, bc_ref, wet_ref, be_ref, traw_ref,
               zq_ref, err_ref, acc_ref):
    i = pl.program_id(0)
    n = pl.num_programs(0)
    t = jax.nn.softplus(traw_ref[...])
    z = z_ref[...]
    zc_t = jax.lax.dot_general(
        wct_ref[...], z, (((0,), (1,)), ((), ())),
        preferred_element_type=jnp.float32) + bc_ref[...]
    z_bound = jnp.tanh(zc_t / t) * _HALF_L
    k = jnp.round(z_bound)
    codes = k * (t * (1.0 / _HALF_W))
    diff = zc_t - codes
    part = jnp.sum(diff * diff, axis=1, keepdims=True)

    @pl.when(i == 0)
    def _init():
        acc_ref[...] = jnp.zeros((3, 1), jnp.float32)

    acc_ref[...] += part

    @pl.when(i == n - 1)
    def _flush():
        err_ref[...] = jnp.full((1, 1, 1), jnp.sum(acc_ref[...]),
                                jnp.float32)

    k_aug = jnp.concatenate(
        [k, jnp.ones((1, k.shape[1]), jnp.float32)], axis=0)
    we_scaled = wet_ref[...] * (t * (1.0 / _HALF_W))
    we_aug = jnp.concatenate([we_scaled, be_ref[...]], axis=0)
    zq_ref[...] = jax.lax.dot_general(
        k_aug.astype(jnp.bfloat16), we_aug.astype(jnp.bfloat16),
        (((0,), (0,)), ((), ())),
        preferred_element_type=jnp.float32)


def kernel(z, W_c, b_c, W_e, b_e, T_raw):
    B, S, D = z.shape
    rows = B * S
    z2 = z.reshape(rows, D)
    blk = 2048
    grid = rows // blk

    zq2, err = pl.pallas_call(
        _fsq_block,
        grid=(grid,),
        in_specs=[
            pl.BlockSpec((blk, D), lambda i: (i, 0)),
            pl.BlockSpec((D, 3), lambda i: (0, 0)),
            pl.BlockSpec((3, 1), lambda i: (0, 0)),
            pl.BlockSpec((3, D), lambda i: (0, 0)),
            pl.BlockSpec((1, D), lambda i: (0, 0)),
            pl.BlockSpec((3, 1), lambda i: (0, 0)),
        ],
        out_specs=[
            pl.BlockSpec((blk, D), lambda i: (i, 0)),
            pl.BlockSpec((1, 1, 1), lambda i: (0, 0, 0)),
        ],
        out_shape=[
            jax.ShapeDtypeStruct((rows, D), jnp.float32),
            jax.ShapeDtypeStruct((1, 1, 1), jnp.float32),
        ],
        scratch_shapes=[pltpu.VMEM((3, 1), jnp.float32)],
        compiler_params=pltpu.CompilerParams(
            dimension_semantics=("parallel",),
        ),
    )(z2, W_c.T, b_c.reshape(3, 1), W_e.T, b_e.reshape(1, D),
      T_raw.reshape(3, 1))

    z_q = zq2.reshape(B, S, D)
    quantization_error = err[0, 0, 0] / (rows * 3)
    return (z_q, quantization_error)
